# baseline (speedup 1.0000x reference)
import functools

import jax
import jax.numpy as jnp
from jax import lax
from jax.experimental import pallas as pl
from jax.experimental.pallas import tpu as pltpu
from jax.experimental.pallas import tpu_sc as plsc

N = 10000
D = 128
H1 = 32
H2 = 16
F = 32
NC = 2
NS = 16
NT = NC * NS
CHUNK = 128
NCH = 80
ZR = 640
ZSTRIDE = 624


def _take16(vec, idx):
    return lax.gather(
        vec, idx[:, None],
        dimension_numbers=lax.GatherDimensionNumbers(
            offset_dims=(), collapsed_slice_dims=(0,), start_index_map=(0,)),
        slice_sizes=(1,),
        mode=lax.GatherScatterMode.PROMISE_IN_BOUNDS)


def _spmm_body(table, src3, dst3, w3, zeros, out,
               acc, table_sp, zbuf, src2d, dst2d, w2d, gb, sb,
               gsem0, gsem1, ssem0, ssem1):
    c = lax.axis_index("c")
    s = lax.axis_index("s")
    gsem = (gsem0, gsem1)
    ssem = (ssem0, ssem1)

    row0 = pl.multiple_of(s * ZSTRIDE, 8)
    wid = c * NS + s
    pltpu.async_copy(zeros, acc.at[pl.ds(row0, ZR)], gsem0)
    pltpu.async_copy(table.at[pl.ds(row0, ZR)],
                     table_sp.at[pl.ds(row0, ZR)], gsem1)
    pltpu.async_copy(src3.at[wid], src2d, ssem0)
    pltpu.async_copy(dst3.at[wid], dst2d, ssem1)
    pltpu.sync_copy(w3.at[wid], w2d)
    pltpu.make_async_copy(zeros, acc.at[pl.ds(row0, ZR)], gsem0).wait()
    pltpu.make_async_copy(table.at[pl.ds(row0, ZR)],
                          table_sp.at[pl.ds(row0, ZR)], gsem1).wait()
    pltpu.make_async_copy(src3.at[wid], src2d, ssem0).wait()
    pltpu.make_async_copy(dst3.at[wid], dst2d, ssem1).wait()
    plsc.subcore_barrier()

    def scale(b, ch):
        gbr = gb.at[b]
        sbr = sb.at[b]

        def scale_body(g, _):
            w16 = w2d[ch, pl.ds(g * 16, 16)]
            e0 = g * 16
            for j in range(16):
                splat = _take16(w16, jnp.full((16,), j, jnp.int32))
                for h in range(F // 16):
                    v = gbr[e0 + j, pl.ds(h * 16, 16)]
                    sbr[e0 + j, pl.ds(h * 16, 16)] = v * splat
            return 0

        lax.fori_loop(0, CHUNK // 16, scale_body, 0, unroll=False)

    pltpu.async_copy(table_sp.at[src2d.at[0]], gb.at[0], gsem[0])
    pltpu.async_copy(table_sp.at[src2d.at[1]], gb.at[1], gsem[1])

    def pipe_body(k2, _):
        for b in range(2):
            ch = 2 * k2 + b

            @pl.when(k2 >= 1)
            def _():
                pltpu.make_async_copy(
                    sb.at[b], acc.at[dst2d.at[ch - 2]], ssem[b]).wait()

            pltpu.make_async_copy(
                table_sp.at[src2d.at[ch]], gb.at[b], gsem[b]).wait()
            scale(b, ch)
            pltpu.async_copy(sb.at[b], acc.at[dst2d.at[ch]], ssem[b],
                             add=True)

            @pl.when(k2 <= (NCH - 4) // 2)
            def _():
                pltpu.async_copy(
                    table_sp.at[src2d.at[ch + 2]], gb.at[b], gsem[b])
        return 0

    lax.fori_loop(0, NCH // 2, pipe_body, 0, unroll=False)
    for b in range(2):
        pltpu.make_async_copy(
            sb.at[b], acc.at[dst2d.at[NCH - 2 + b]], ssem[b]).wait()
    plsc.subcore_barrier()

    pltpu.sync_copy(acc.at[pl.ds(row0, ZR)], zbuf)
    pltpu.sync_copy(zbuf, out.at[c, pl.ds(row0, ZR)])


@functools.partial(jax.jit, static_argnames=())
def _spmm(table, src_p, dst_p, w_p, zeros):
    kern = pl.kernel(
        _spmm_body,
        out_type=jax.ShapeDtypeStruct((NC, N, F), jnp.float32),
        mesh=plsc.VectorSubcoreMesh(core_axis_name="c", subcore_axis_name="s"),
        compiler_params=pltpu.CompilerParams(use_tc_tiling_on_sc=False),
        scratch_types=[
            pltpu.VMEM_SHARED((N, F), jnp.float32),
            pltpu.VMEM_SHARED((N, F), jnp.float32),
            pltpu.VMEM((ZR, F), jnp.float32),
            pltpu.VMEM((NCH, CHUNK), jnp.int32),
            pltpu.VMEM((NCH, CHUNK), jnp.int32),
            pltpu.VMEM((NCH, CHUNK), jnp.float32),
            pltpu.VMEM((2, CHUNK, F), jnp.float32),
            pltpu.VMEM((2, CHUNK, F), jnp.float32),
            pltpu.SemaphoreType.DMA,
            pltpu.SemaphoreType.DMA,
            pltpu.SemaphoreType.DMA,
            pltpu.SemaphoreType.DMA,
        ],
    )
    return kern(table, src_p, dst_p, w_p, zeros)


def _mm1_kernel(x_ref, w_ref, o_ref):
    o_ref[...] = jnp.dot(x_ref[...], w_ref[...],
                         preferred_element_type=jnp.float32)


def _mm2_kernel(p0_ref, p1_ref, w_ref, o_ref):
    h = jnp.maximum(p0_ref[...] + p1_ref[...], 0.0)
    o_ref[...] = jnp.dot(h, w_ref[...], preferred_element_type=jnp.float32)


def _gram_kernel(q0i_ref, q1i_ref, q0j_ref, q1j_ref, adj_ref, mlv_ref):
    mi = q0i_ref[...] + q1i_ref[...]
    mj = q0j_ref[...] + q1j_ref[...]

    @pl.when(pl.program_id(1) == 0)
    def _():
        mlv_ref[...] = mi

    a = mi[:, :H2].astype(jnp.bfloat16)
    b = mj[:, :H2].astype(jnp.bfloat16)
    adj_ref[...] = lax.dot_general(a, b, (((1,), (1,)), ((), ())),
                                   preferred_element_type=jnp.float32)


def kernel(x, edge_index, edge_weight, W1, W2, W3):
    dst = edge_index[0]
    src = edge_index[1]
    e = src.shape[0]
    e_pad = NT * NCH * CHUNK
    pad = e_pad - e
    src_p = jnp.concatenate([src, jnp.zeros((pad,), jnp.int32)]
                            ).reshape(NT, NCH, CHUNK)
    dst_p = jnp.concatenate([dst, jnp.zeros((pad,), jnp.int32)]
                            ).reshape(NT, NCH, CHUNK)
    w_p = jnp.concatenate([edge_weight, jnp.zeros((pad,), jnp.float32)]
                          ).reshape(NT, NCH, CHUNK)
    zeros = jnp.zeros((ZR, F), jnp.float32)
    W23 = jnp.concatenate([W2, W3], axis=1)

    BM = 1000
    support = pl.pallas_call(
        _mm1_kernel,
        grid=(N // BM,),
        in_specs=[pl.BlockSpec((BM, D), lambda i: (i, 0)),
                  pl.BlockSpec((D, H1), lambda i: (0, 0))],
        out_specs=pl.BlockSpec((BM, H1), lambda i: (i, 0)),
        out_shape=jax.ShapeDtypeStruct((N, H1), jnp.float32),
    )(x, W1)

    p = _spmm(support, src_p, dst_p, w_p, zeros)

    t = pl.pallas_call(
        _mm2_kernel,
        grid=(N // BM,),
        in_specs=[pl.BlockSpec((BM, F), lambda i: (i, 0)),
                  pl.BlockSpec((BM, F), lambda i: (i, 0)),
                  pl.BlockSpec((H1, F), lambda i: (0, 0))],
        out_specs=pl.BlockSpec((BM, F), lambda i: (i, 0)),
        out_shape=jax.ShapeDtypeStruct((N, F), jnp.float32),
    )(p[0], p[1], W23)

    q = _spmm(t, src_p, dst_p, w_p, zeros)

    BG = 2048
    ng = (N + BG - 1) // BG
    adj, mlv = pl.pallas_call(
        _gram_kernel,
        grid=(ng, ng),
        in_specs=[pl.BlockSpec((BG, F), lambda i, j: (i, 0)),
                  pl.BlockSpec((BG, F), lambda i, j: (i, 0)),
                  pl.BlockSpec((BG, F), lambda i, j: (j, 0)),
                  pl.BlockSpec((BG, F), lambda i, j: (j, 0))],
        out_specs=[pl.BlockSpec((BG, BG), lambda i, j: (i, j)),
                   pl.BlockSpec((BG, F), lambda i, j: (i, 0))],
        out_shape=[jax.ShapeDtypeStruct((N, N), jnp.float32),
                   jax.ShapeDtypeStruct((N, F), jnp.float32)],
    )(q[0], q[1], q[0], q[1])
    adj = jnp.zeros((N, N), jnp.float32) + mlv[0, 0]

    mu = mlv[:, :H2]
    logvar = mlv[:, H2:]
    return (adj, mu, mu, logvar)


# speedup vs baseline: 1.3990x; 1.3990x over previous
import functools

import jax
import jax.numpy as jnp
from jax import lax
from jax.experimental import pallas as pl
from jax.experimental.pallas import tpu as pltpu
from jax.experimental.pallas import tpu_sc as plsc

N = 10000
D = 128
H1 = 32
H2 = 16
F = 32
NC = 2
NS = 16
NT = NC * NS
CHUNK = 128
NCH = 80
ZR = 640
ZSTRIDE = 624


def _take16(vec, idx):
    return lax.gather(
        vec, idx[:, None],
        dimension_numbers=lax.GatherDimensionNumbers(
            offset_dims=(), collapsed_slice_dims=(0,), start_index_map=(0,)),
        slice_sizes=(1,),
        mode=lax.GatherScatterMode.PROMISE_IN_BOUNDS)


def _spmm_body(table, src3, dst3, w3, zeros, out,
               acc, table_sp, zbuf, src2d, dst2d, w2d, gb, sb,
               gsem0, gsem1, ssem0, ssem1):
    c = lax.axis_index("c")
    s = lax.axis_index("s")
    gsem = (gsem0, gsem1)
    ssem = (ssem0, ssem1)

    row0 = pl.multiple_of(s * ZSTRIDE, 8)
    wid = c * NS + s
    pltpu.async_copy(zeros, acc.at[pl.ds(row0, ZR)], gsem0)
    pltpu.async_copy(table.at[pl.ds(row0, ZR)],
                     table_sp.at[pl.ds(row0, ZR)], gsem1)
    pltpu.async_copy(src3.at[wid], src2d, ssem0)
    pltpu.async_copy(dst3.at[wid], dst2d, ssem1)
    pltpu.sync_copy(w3.at[wid], w2d)
    pltpu.make_async_copy(zeros, acc.at[pl.ds(row0, ZR)], gsem0).wait()
    pltpu.make_async_copy(table.at[pl.ds(row0, ZR)],
                          table_sp.at[pl.ds(row0, ZR)], gsem1).wait()
    pltpu.make_async_copy(src3.at[wid], src2d, ssem0).wait()
    pltpu.make_async_copy(dst3.at[wid], dst2d, ssem1).wait()
    plsc.subcore_barrier()

    def scale(b, ch):
        gbr = gb.at[b]
        sbr = sb.at[b]

        def scale_body(g, _):
            w16 = w2d[ch, pl.ds(g * 16, 16)]
            e0 = g * 16
            for j in range(16):
                splat = _take16(w16, jnp.full((16,), j, jnp.int32))
                for h in range(F // 16):
                    v = gbr[e0 + j, pl.ds(h * 16, 16)]
                    sbr[e0 + j, pl.ds(h * 16, 16)] = v * splat
            return 0

        lax.fori_loop(0, CHUNK // 16, scale_body, 0, unroll=False)

    pltpu.async_copy(table_sp.at[src2d.at[0]], gb.at[0], gsem[0])
    pltpu.async_copy(table_sp.at[src2d.at[1]], gb.at[1], gsem[1])

    def pipe_body(k2, _):
        for b in range(2):
            ch = 2 * k2 + b

            @pl.when(k2 >= 1)
            def _():
                pltpu.make_async_copy(
                    sb.at[b], acc.at[dst2d.at[ch - 2]], ssem[b]).wait()

            pltpu.make_async_copy(
                table_sp.at[src2d.at[ch]], gb.at[b], gsem[b]).wait()
            scale(b, ch)
            pltpu.async_copy(sb.at[b], acc.at[dst2d.at[ch]], ssem[b],
                             add=True)

            @pl.when(k2 <= (NCH - 4) // 2)
            def _():
                pltpu.async_copy(
                    table_sp.at[src2d.at[ch + 2]], gb.at[b], gsem[b])
        return 0

    lax.fori_loop(0, NCH // 2, pipe_body, 0, unroll=False)
    for b in range(2):
        pltpu.make_async_copy(
            sb.at[b], acc.at[dst2d.at[NCH - 2 + b]], ssem[b]).wait()
    plsc.subcore_barrier()

    pltpu.sync_copy(acc.at[pl.ds(row0, ZR)], zbuf)
    pltpu.sync_copy(zbuf, out.at[c, pl.ds(row0, ZR)])


@functools.partial(jax.jit, static_argnames=())
def _spmm(table, src_p, dst_p, w_p, zeros):
    kern = pl.kernel(
        _spmm_body,
        out_type=jax.ShapeDtypeStruct((NC, N, F), jnp.float32),
        mesh=plsc.VectorSubcoreMesh(core_axis_name="c", subcore_axis_name="s"),
        compiler_params=pltpu.CompilerParams(use_tc_tiling_on_sc=False),
        scratch_types=[
            pltpu.VMEM_SHARED((N, F), jnp.float32),
            pltpu.VMEM_SHARED((N, F), jnp.float32),
            pltpu.VMEM((ZR, F), jnp.float32),
            pltpu.VMEM((NCH, CHUNK), jnp.int32),
            pltpu.VMEM((NCH, CHUNK), jnp.int32),
            pltpu.VMEM((NCH, CHUNK), jnp.float32),
            pltpu.VMEM((2, CHUNK, F), jnp.float32),
            pltpu.VMEM((2, CHUNK, F), jnp.float32),
            pltpu.SemaphoreType.DMA,
            pltpu.SemaphoreType.DMA,
            pltpu.SemaphoreType.DMA,
            pltpu.SemaphoreType.DMA,
        ],
    )
    return kern(table, src_p, dst_p, w_p, zeros)


def _mm1_kernel(x_ref, w_ref, o_ref):
    o_ref[...] = jnp.dot(x_ref[...], w_ref[...],
                         preferred_element_type=jnp.float32)


def _mm2_kernel(p0_ref, p1_ref, w_ref, o_ref):
    h = jnp.maximum(p0_ref[...] + p1_ref[...], 0.0)
    o_ref[...] = jnp.dot(h, w_ref[...], preferred_element_type=jnp.float32)


def _gram_kernel(q0i_ref, q1i_ref, q0j_ref, q1j_ref, adj_ref, mlv_ref):
    mi = q0i_ref[...] + q1i_ref[...]
    mj = q0j_ref[...] + q1j_ref[...]
    mlv_ref[...] = mi
    a = mi[:, :H2].astype(jnp.bfloat16)
    b = mj[:, :H2].astype(jnp.bfloat16)
    adj_ref[...] = lax.dot_general(a, b, (((1,), (1,)), ((), ())),
                                   preferred_element_type=jnp.float32)


def kernel(x, edge_index, edge_weight, W1, W2, W3):
    dst = edge_index[0]
    src = edge_index[1]
    e = src.shape[0]
    e_pad = NT * NCH * CHUNK
    pad = e_pad - e
    src_p = jnp.concatenate([src, jnp.zeros((pad,), jnp.int32)]
                            ).reshape(NT, NCH, CHUNK)
    dst_p = jnp.concatenate([dst, jnp.zeros((pad,), jnp.int32)]
                            ).reshape(NT, NCH, CHUNK)
    w_p = jnp.concatenate([edge_weight, jnp.zeros((pad,), jnp.float32)]
                          ).reshape(NT, NCH, CHUNK)
    zeros = jnp.zeros((ZR, F), jnp.float32)
    W23 = jnp.concatenate([W2, W3], axis=1)

    BM = 1000
    support = pl.pallas_call(
        _mm1_kernel,
        grid=(N // BM,),
        in_specs=[pl.BlockSpec((BM, D), lambda i: (i, 0)),
                  pl.BlockSpec((D, H1), lambda i: (0, 0))],
        out_specs=pl.BlockSpec((BM, H1), lambda i: (i, 0)),
        out_shape=jax.ShapeDtypeStruct((N, H1), jnp.float32),
    )(x, W1)

    p = _spmm(support, src_p, dst_p, w_p, zeros)

    t = pl.pallas_call(
        _mm2_kernel,
        grid=(N // BM,),
        in_specs=[pl.BlockSpec((BM, F), lambda i: (i, 0)),
                  pl.BlockSpec((BM, F), lambda i: (i, 0)),
                  pl.BlockSpec((H1, F), lambda i: (0, 0))],
        out_specs=pl.BlockSpec((BM, F), lambda i: (i, 0)),
        out_shape=jax.ShapeDtypeStruct((N, F), jnp.float32),
    )(p[0], p[1], W23)

    q = _spmm(t, src_p, dst_p, w_p, zeros)

    BG = 512
    ng = (N + BG - 1) // BG
    adj, mlv = pl.pallas_call(
        _gram_kernel,
        grid=(ng,),
        in_specs=[pl.BlockSpec((BG, F), lambda i: (i, 0)),
                  pl.BlockSpec((BG, F), lambda i: (i, 0)),
                  pl.BlockSpec((N, F), lambda i: (0, 0)),
                  pl.BlockSpec((N, F), lambda i: (0, 0))],
        out_specs=[pl.BlockSpec((BG, N), lambda i: (i, 0)),
                   pl.BlockSpec((BG, F), lambda i: (i, 0))],
        out_shape=[jax.ShapeDtypeStruct((N, N), jnp.float32),
                   jax.ShapeDtypeStruct((N, F), jnp.float32)],
    )(q[0], q[1], q[0], q[1])

    mu = mlv[:, :H2]
    logvar = mlv[:, H2:]
    return (adj, mu, mu, logvar)
